# baseline (speedup 1.0000x reference)
import functools

import jax
import jax.numpy as jnp
from jax import lax
from jax.experimental import pallas as pl
from jax.experimental.pallas import tpu as pltpu
from jax.experimental.pallas import tpu_sc as plsc

VOCAB = 1000000
EMBED_DIM = 64
BATCH = 16384
HIST = 50
HIST_PAD = 64
HIDDEN = 256
NUM_CLASSES = 1000

_INFO = plsc.get_sparse_core_info()
NC = _INFO.num_cores
NS = _INFO.num_subcores
NW = NC * NS
BPW = BATCH // NW
CHUNK_IDX = 128
ROWS_PER_CHUNK = CHUNK_IDX // HIST_PAD
NCHUNK = BPW * HIST_PAD // CHUNK_IDX
NBUF = 2


def _sc_pool(x_pad, table):
    mesh = plsc.VectorSubcoreMesh(core_axis_name="c", subcore_axis_name="s")

    @functools.partial(
        pl.kernel,
        out_type=jax.ShapeDtypeStruct((BATCH, EMBED_DIM), jnp.float32),
        mesh=mesh,
        scratch_types=[
            pltpu.VMEM((NCHUNK, CHUNK_IDX), jnp.int32),
            pltpu.VMEM((NBUF, CHUNK_IDX, EMBED_DIM), jnp.float32),
            pltpu.VMEM((BPW, EMBED_DIM), jnp.float32),
            [pltpu.SemaphoreType.DMA] * NBUF,
        ],
        compiler_params=pltpu.CompilerParams(use_tc_tiling_on_sc=False),
    )
    def k(x_hbm, table_hbm, out_hbm, idx_v, rows_v, pooled_v, sems):
        s = lax.axis_index("s")
        c = lax.axis_index("c")
        wid = s * NC + c

        pltpu.sync_copy(x_hbm.at[wid], idx_v)

        for b in range(NBUF):
            pltpu.async_copy(table_hbm.at[idx_v.at[b]], rows_v.at[b], sems[b])

        def body(i, carry):
            del carry
            for b in range(NBUF):
                ck = i * NBUF + b
                pltpu.make_async_copy(
                    table_hbm.at[idx_v.at[b]], rows_v.at[b], sems[b]
                ).wait()
                for r in range(ROWS_PER_CHUNK):
                    accs = [
                        rows_v[b, r * HIST_PAD, pl.ds(cc * 16, 16)]
                        for cc in range(EMBED_DIM // 16)
                    ]
                    for j in range(1, HIST):
                        for cc in range(EMBED_DIM // 16):
                            accs[cc] = accs[cc] + rows_v[
                                b, r * HIST_PAD + j, pl.ds(cc * 16, 16)
                            ]
                    prow = ck * ROWS_PER_CHUNK + r
                    for cc in range(EMBED_DIM // 16):
                        pooled_v[prow, pl.ds(cc * 16, 16)] = accs[cc] * (
                            1.0 / HIST
                        )
                @pl.when(ck + NBUF < NCHUNK)
                def _():
                    pltpu.async_copy(
                        table_hbm.at[idx_v.at[ck + NBUF]], rows_v.at[b], sems[b]
                    )
            return 0

        lax.fori_loop(0, NCHUNK // NBUF, body, 0)

        pltpu.sync_copy(pooled_v, out_hbm.at[pl.ds(wid * BPW, BPW)])

    return k(x_pad, table)


def _tc_mlp(pooled, W1, b1, W2, b2):
    BM = 2048

    def body(x_ref, w1_ref, b1_ref, w2_ref, b2_ref, o_ref):
        h = jnp.dot(x_ref[...], w1_ref[...], preferred_element_type=jnp.float32)
        h = jnp.maximum(h + b1_ref[...], 0.0)
        o_ref[...] = (
            jnp.dot(h, w2_ref[...], preferred_element_type=jnp.float32)
            + b2_ref[...]
        )

    return pl.pallas_call(
        body,
        grid=(BATCH // BM,),
        in_specs=[
            pl.BlockSpec((BM, EMBED_DIM), lambda i: (i, 0)),
            pl.BlockSpec((EMBED_DIM, HIDDEN), lambda i: (0, 0)),
            pl.BlockSpec((1, HIDDEN), lambda i: (0, 0)),
            pl.BlockSpec((HIDDEN, NUM_CLASSES), lambda i: (0, 0)),
            pl.BlockSpec((1, NUM_CLASSES), lambda i: (0, 0)),
        ],
        out_specs=pl.BlockSpec((BM, NUM_CLASSES), lambda i: (i, 0)),
        out_shape=jax.ShapeDtypeStruct((BATCH, NUM_CLASSES), jnp.float32),
    )(pooled, W1, b1.reshape(1, HIDDEN), W2, b2.reshape(1, NUM_CLASSES))


def kernel(x, table, W1, b1, W2, b2):
    x_pad = jnp.pad(x.astype(jnp.int32), ((0, 0), (0, HIST_PAD - HIST)))
    x_pad = x_pad.reshape(NW, NCHUNK, CHUNK_IDX)
    pooled = _sc_pool(x_pad, table)
    return _tc_mlp(pooled, W1, b1, W2, b2)
